# baseline (speedup 1.0000x reference)
import functools
import numpy as np
import jax
import jax.numpy as jnp
from jax import lax
from jax.experimental import pallas as pl
from jax.experimental.pallas import tpu as pltpu
from jax.experimental.pallas import tpu_sc as plsc

K = 80
G_EPS = 1e-5
N = 2048
TN = 512
TNB = 128
NEG_INF = float('-inf')


def _knn_body(xt_ref, xn_ref, idx_ref, d_ref, iscr_ref, key_ref):
    a = xt_ref[0]
    xn = xn_ref[0]
    t = pl.program_id(1)
    a_cn = xn_ref[0, :, pl.ds(t * TN, TN)]
    ip = lax.dot_general(a_cn, xn, (((0,), (0,)), ((), ())),
                         preferred_element_type=jnp.float32)
    xx_t = jnp.sum(a * a, axis=1, keepdims=True)
    xx_r = jnp.sum(xn * xn, axis=0, keepdims=True)
    d_ref[...] = 2.0 * ip - xx_t - xx_r

    R = a.shape[0]
    n = xn.shape[1]
    kb = lax.bitcast_convert_type(d_ref[...], jnp.int32)
    key = kb ^ (jnp.right_shift(kb, 31) & jnp.int32(0x7FFFFFFF))
    key_ref[...] = key
    lo = jnp.min(key, axis=1, keepdims=True)
    hi = jnp.max(key, axis=1, keepdims=True)

    def bis(_, lohi):
        lo, hi = lohi
        mid = lo + jnp.right_shift(hi - lo + 1, 1)
        cnt = jnp.sum((key_ref[...] >= mid).astype(jnp.int32), axis=1,
                      keepdims=True)
        ge = cnt >= K
        return (jnp.where(ge, mid, lo), jnp.where(ge, hi, mid - 1))

    lo, hi = lax.fori_loop(0, 31, bis, (lo, hi))
    tau = lo

    key3 = key_ref[...].reshape(R, n // 16, 16)
    j3 = lax.broadcasted_iota(jnp.int32, (R, n // 16, 16), 2)
    t3 = tau[:, :, None]
    gbits0 = jnp.sum(jnp.left_shift((key3 > t3).astype(jnp.int32), j3), axis=2)
    ebits0 = jnp.sum(jnp.left_shift((key3 == t3).astype(jnp.int32), j3),
                     axis=2)

    NL = n // 16
    liota = lax.broadcasted_iota(jnp.int32, (R, NL), 1)
    kiota = lax.broadcasted_iota(jnp.int32, (R, K), 1)
    iscr_ref[...] = jnp.zeros((R, K), jnp.int32)

    def body(k, ge):
        gbits, ebits = ge
        has_g = jnp.max((gbits != 0).astype(jnp.int32), axis=1,
                        keepdims=True) > 0
        cur = jnp.where(has_g, gbits, ebits)
        nz = cur != 0
        lane = jnp.min(jnp.where(nz, liota, NL), axis=1, keepdims=True)
        onlane = liota == lane
        v = jnp.sum(jnp.where(onlane, cur, 0), axis=1, keepdims=True)
        low = v & (-v)
        fe = lax.bitcast_convert_type(low.astype(jnp.float32), jnp.int32)
        bitpos = jnp.right_shift(fe, 23) - 127
        am = lane * 16 + bitpos
        iscr_ref[...] = iscr_ref[...] + jnp.where(kiota == k, am, 0)
        clr = jnp.where(onlane, low, 0)
        gbits = jnp.where(has_g, gbits ^ clr, gbits)
        ebits = jnp.where(has_g, ebits, ebits ^ clr)
        return (gbits, ebits)

    lax.fori_loop(0, K, body, (gbits0, ebits0))
    idx_ref[0] = iscr_ref[...] + pl.program_id(0) * N


def _knn(xt, xn):
    B, n, C = xt.shape
    return pl.pallas_call(
        _knn_body,
        grid=(B, n // TN),
        in_specs=[
            pl.BlockSpec((1, TN, C), lambda b, t: (b, t, 0)),
            pl.BlockSpec((1, C, n), lambda b, t: (b, 0, 0)),
        ],
        out_specs=pl.BlockSpec((1, TN, K), lambda b, t: (b, t, 0)),
        out_shape=jax.ShapeDtypeStruct((B, n, K), jnp.int32),
        scratch_shapes=[pltpu.VMEM((TN, n), jnp.float32),
                        pltpu.VMEM((TN, K), jnp.int32),
                        pltpu.VMEM((TN, n), jnp.int32)],
    )(xt, xn)


def _sc_gather(table, idxflat):
    M = idxflat.shape[0]
    C = table.shape[1]
    W = 128
    idx2 = idxflat.reshape(1, M)
    mesh = plsc.VectorSubcoreMesh(core_axis_name="core",
                                  subcore_axis_name="subcore")

    @functools.partial(
        pl.kernel,
        out_type=jax.ShapeDtypeStruct((M, C), jnp.float32),
        mesh=mesh,
        compiler_params=pltpu.CompilerParams(use_tc_tiling_on_sc=False),
    )
    def kern(x_hbm, i_hbm, o_hbm):
        def body(i_vmem, o_vmem):
            pltpu.sync_copy(x_hbm.at[i_vmem.at[0]], o_vmem)

        pltpu.emit_pipeline(
            body,
            grid=(M // W,),
            in_specs=[pl.BlockSpec((1, W), index_map=lambda i: (0, i))],
            out_specs=[pl.BlockSpec((W, C), index_map=lambda i: (i, 0))],
            core_axis_name=("core", "subcore"),
            dimension_semantics=(pltpu.PARALLEL,),
        )(i_hbm, o_hbm)

    return kern(table, idx2)


def _econv_body(g_ref, xt_ref, w_ref, mx_ref, mn_ref, sp_ref, qp_ref):
    g2 = g_ref[0]
    xb = xt_ref[0]
    C = xb.shape[1]
    xbro = jnp.broadcast_to(xb[:, None, :], (TNB, K, C)).reshape(TNB * K, C)
    fcat = jnp.concatenate([g2 - xbro, xbro], axis=1)
    t = jnp.dot(fcat, w_ref[...], preferred_element_type=jnp.float32)
    O = t.shape[1]
    t3 = t.reshape(TNB, K, O)
    mx_ref[0] = jnp.max(t3, axis=1)
    mn_ref[0] = jnp.min(t3, axis=1)
    sp_ref[0, 0] = jnp.sum(t, axis=0, keepdims=True)
    qp_ref[0, 0] = jnp.sum(t * t, axis=0, keepdims=True)


def _econv(g, xt, w2T):
    B, n, C = xt.shape
    O = w2T.shape[1]
    NT = n // TNB
    return pl.pallas_call(
        _econv_body,
        grid=(B, NT),
        in_specs=[
            pl.BlockSpec((1, TNB * K, C), lambda b, t: (b, t, 0)),
            pl.BlockSpec((1, TNB, C), lambda b, t: (b, t, 0)),
            pl.BlockSpec((2 * C, O), lambda b, t: (0, 0)),
        ],
        out_specs=[
            pl.BlockSpec((1, TNB, O), lambda b, t: (b, t, 0)),
            pl.BlockSpec((1, TNB, O), lambda b, t: (b, t, 0)),
            pl.BlockSpec((1, 1, 1, O), lambda b, t: (b, t, 0, 0)),
            pl.BlockSpec((1, 1, 1, O), lambda b, t: (b, t, 0, 0)),
        ],
        out_shape=[jax.ShapeDtypeStruct((B, n, O), jnp.float32),
                   jax.ShapeDtypeStruct((B, n, O), jnp.float32),
                   jax.ShapeDtypeStruct((B, NT, 1, O), jnp.float32),
                   jax.ShapeDtypeStruct((B, NT, 1, O), jnp.float32)],
    )(g, xt, w2T)


def _group_norm_consts(s, q, cnt, G):
    O = s.shape[1]
    W = O // G
    mc, vc = [], []
    for g in range(G):
        sg = jnp.sum(s[:, g * W:(g + 1) * W], axis=1, keepdims=True) / cnt
        qg = jnp.sum(q[:, g * W:(g + 1) * W], axis=1, keepdims=True) / cnt
        mc.append(jnp.broadcast_to(sg, (1, W)))
        vc.append(jnp.broadcast_to(qg - sg * sg, (1, W)))
    return jnp.concatenate(mc, axis=1), jnp.concatenate(vc, axis=1)


def _fin_body(mx_ref, mn_ref, sp_ref, qp_ref, g_ref, b_ref,
              xt_ref, xn_ref, *, cnt, slope, groups):
    s = jnp.sum(sp_ref[0, :, 0, :], axis=0, keepdims=True)
    q = jnp.sum(qp_ref[0, :, 0, :], axis=0, keepdims=True)
    mean_c, var_c = _group_norm_consts(s, q, cnt, groups)
    den = jnp.sqrt(var_c + G_EPS)
    gam = g_ref[...]
    bet = b_ref[...]
    sel_v = jnp.where(gam >= 0, mx_ref[0], mn_ref[0])
    xh = (sel_v - mean_c) / den * gam + bet
    xo = jnp.where(xh >= 0, xh, slope * xh)
    xt_ref[0] = xo
    xn_ref[0] = xo.T


def _finalize(mx, mn, sp, qp, gam, bet, groups, slope):
    B, n, O = mx.shape
    cnt = float((O // groups) * n * K)
    body = functools.partial(_fin_body, cnt=cnt, slope=slope, groups=groups)
    NT = sp.shape[1]
    assert sp.shape == (B, NT, 1, O)
    return pl.pallas_call(
        body,
        grid=(B,),
        in_specs=[
            pl.BlockSpec((1, n, O), lambda b: (b, 0, 0)),
            pl.BlockSpec((1, n, O), lambda b: (b, 0, 0)),
            pl.BlockSpec((1, NT, 1, O), lambda b: (b, 0, 0, 0)),
            pl.BlockSpec((1, NT, 1, O), lambda b: (b, 0, 0, 0)),
            pl.BlockSpec((1, O), lambda b: (0, 0)),
            pl.BlockSpec((1, O), lambda b: (0, 0)),
        ],
        out_specs=[
            pl.BlockSpec((1, n, O), lambda b: (b, 0, 0)),
            pl.BlockSpec((1, O, n), lambda b: (b, 0, 0)),
        ],
        out_shape=[jax.ShapeDtypeStruct((B, n, O), jnp.float32),
                   jax.ShapeDtypeStruct((B, O, n), jnp.float32)],
    )(mx, mn, sp, qp, gam.reshape(1, O), bet.reshape(1, O))


def _edge_layer(xt, xn, w2T, gam, bet, groups=2, slope=0.2):
    B, n, C = xt.shape
    idx = _knn(xt, xn)
    g = _sc_gather(xt.reshape(B * n, C), idx.reshape(B * n * K))
    g = g.reshape(B, n * K, C)
    mx, mn, sp, qp = _econv(g, xt, w2T)
    return _finalize(mx, mn, sp, qp, gam, bet, groups, slope)


def _gn_rows(t, gam, bet, groups, cnt):
    col_s = jnp.sum(t, axis=0, keepdims=True)
    col_q = jnp.sum(t * t, axis=0, keepdims=True)
    mean_c, var_c = _group_norm_consts(col_s, col_q, cnt, groups)
    return (t - mean_c) / jnp.sqrt(var_c + G_EPS) * gam + bet


def _head_body(x1_ref, x2_ref, x3_ref,
               mlp1_ref, mlp1b_ref, gnm_g_ref, gnm_b_ref,
               c1_ref, c1b_ref, bn1_g_ref, bn1_b_ref,
               c2_ref, c2b_ref, bn2_g_ref, bn2_b_ref,
               s1_ref, s1b_ref, bns_g_ref, bns_b_ref,
               s2_ref, s2b_ref,
               p1_ref, p1b_ref, bnp_g_ref, bnp_b_ref,
               p2_ref, p2b_ref,
               emb_ref, lp_ref):
    n = x1_ref.shape[1]
    xf = jnp.concatenate([x1_ref[0], x2_ref[0], x3_ref[0]], axis=1)
    h = jnp.dot(xf, mlp1_ref[...], preferred_element_type=jnp.float32) \
        + mlp1b_ref[...]
    h = _gn_rows(h, gnm_g_ref[...], gnm_b_ref[...], 8, float(n * 128))
    h = jnp.maximum(h, 0.0)
    x4 = jnp.max(h, axis=0, keepdims=True)
    hh = jnp.concatenate([jnp.broadcast_to(x4, (n, 1024)), xf], axis=1)
    h1 = jnp.dot(hh, c1_ref[...], preferred_element_type=jnp.float32) \
        + c1b_ref[...]
    h1 = _gn_rows(h1, bn1_g_ref[...], bn1_b_ref[...], 8, float(n * 64))
    h1 = jnp.maximum(h1, 0.0)
    xa = jnp.dot(h1, c2_ref[...], preferred_element_type=jnp.float32) \
        + c2b_ref[...]
    xa = _gn_rows(xa, bn2_g_ref[...], bn2_b_ref[...], 4, float(n * 64))
    xa = jnp.maximum(xa, 0.0)
    e = jnp.dot(xa, s1_ref[...], preferred_element_type=jnp.float32) \
        + s1b_ref[...]
    e = _gn_rows(e, bns_g_ref[...], bns_b_ref[...], 4, float(n * 64))
    e = jnp.maximum(e, 0.0)
    emb_ref[0] = jnp.dot(e, s2_ref[...], preferred_element_type=jnp.float32) \
        + s2b_ref[...]
    q = jnp.dot(xa, p1_ref[...], preferred_element_type=jnp.float32) \
        + p1b_ref[...]
    q = _gn_rows(q, bnp_g_ref[...], bnp_b_ref[...], 4, float(n * 64))
    q = jnp.maximum(q, 0.0)
    logits = jnp.dot(q, p2_ref[...], preferred_element_type=jnp.float32) \
        + p2b_ref[...]
    lm = jnp.max(logits, axis=1, keepdims=True)
    shifted = logits - lm
    lp_ref[0] = shifted - jnp.log(jnp.sum(jnp.exp(shifted), axis=1,
                                          keepdims=True))


def _head(x1t, x2t, x3t, p):
    B, n, _ = x1t.shape
    r = lambda v: v.reshape(1, -1)
    const = lambda shp: pl.BlockSpec(shp, lambda b: tuple(0 for _ in shp))
    args = [
        p['mlp1_w'].T, r(p['mlp1_b']), r(p['gnm_g']), r(p['gnm_b']),
        p['c1_w'].T, r(p['c1_b']), r(p['bn1_g']), r(p['bn1_b']),
        p['c2_w'].T, r(p['c2_b']), r(p['bn2_g']), r(p['bn2_b']),
        p['seg1_w'].T, r(p['seg1_b']), r(p['bnseg_g']), r(p['bnseg_b']),
        p['seg2_w'].T, r(p['seg2_b']),
        p['prim1_w'].T, r(p['prim1_b']), r(p['bnprim_g']), r(p['bnprim_b']),
        p['prim2_w'].T, r(p['prim2_b']),
    ]
    in_specs = [
        pl.BlockSpec((1, n, 64), lambda b: (b, 0, 0)),
        pl.BlockSpec((1, n, 64), lambda b: (b, 0, 0)),
        pl.BlockSpec((1, n, 128), lambda b: (b, 0, 0)),
    ] + [const(a.shape) for a in args]
    return pl.pallas_call(
        _head_body,
        grid=(B,),
        in_specs=in_specs,
        out_specs=[
            pl.BlockSpec((1, n, 50), lambda b: (b, 0, 0)),
            pl.BlockSpec((1, n, 8), lambda b: (b, 0, 0)),
        ],
        out_shape=[jax.ShapeDtypeStruct((B, n, 50), jnp.float32),
                   jax.ShapeDtypeStruct((B, n, 8), jnp.float32)],
    )(x1t, x2t, x3t, *args)


def kernel(x, conv1_w, gn1_g, gn1_b, conv2_w, gn2_g, gn2_b, conv3_w, gn3_g,
           gn3_b, mlp1_w, mlp1_b, gnm_g, gnm_b, c1_w, c1_b, bn1_g, bn1_b,
           c2_w, c2_b, bn2_g, bn2_b, seg1_w, seg1_b, bnseg_g, bnseg_b,
           seg2_w, seg2_b, prim1_w, prim1_b, bnprim_g, bnprim_b,
           prim2_w, prim2_b):
    B, C0, n = x.shape
    CP = 16
    xn0 = jnp.concatenate([x, jnp.zeros((B, CP - C0, n), jnp.float32)], axis=1)
    xt0 = jnp.transpose(xn0, (0, 2, 1))
    zpad = jnp.zeros((64, CP - C0), jnp.float32)
    w1 = jnp.concatenate([conv1_w[:, :C0], zpad,
                          conv1_w[:, C0:], zpad], axis=1).T
    x1t, x1n = _edge_layer(xt0, xn0, w1, gn1_g, gn1_b)
    x2t, x2n = _edge_layer(x1t, x1n, conv2_w.T, gn2_g, gn2_b)
    x3t, _ = _edge_layer(x2t, x2n, conv3_w.T, gn3_g, gn3_b)
    p = dict(mlp1_w=mlp1_w, mlp1_b=mlp1_b, gnm_g=gnm_g, gnm_b=gnm_b,
             c1_w=c1_w, c1_b=c1_b, bn1_g=bn1_g, bn1_b=bn1_b,
             c2_w=c2_w, c2_b=c2_b, bn2_g=bn2_g, bn2_b=bn2_b,
             seg1_w=seg1_w, seg1_b=seg1_b, bnseg_g=bnseg_g, bnseg_b=bnseg_b,
             seg2_w=seg2_w, seg2_b=seg2_b,
             prim1_w=prim1_w, prim1_b=prim1_b, bnprim_g=bnprim_g,
             bnprim_b=bnprim_b, prim2_w=prim2_w, prim2_b=prim2_b)
    emb, lp = _head(x1t, x2t, x3t, p)
    return (jnp.transpose(emb, (0, 2, 1)), jnp.transpose(lp, (0, 2, 1)))


# speedup vs baseline: 1.2950x; 1.2950x over previous
import functools
import numpy as np
import jax
import jax.numpy as jnp
from jax import lax
from jax.experimental import pallas as pl
from jax.experimental.pallas import tpu as pltpu
from jax.experimental.pallas import tpu_sc as plsc

K = 80
G_EPS = 1e-5
N = 2048
TN = 1024
TNB = 128
NEG_INF = float('-inf')


def _knn_body(xt_ref, xn_ref, idx_ref, d_ref, iscr_ref):
    a = xt_ref[0]
    xn = xn_ref[0]
    t = pl.program_id(1)
    a_cn = xn_ref[0, :, pl.ds(t * TN, TN)]
    ip = lax.dot_general(a_cn, xn, (((0,), (0,)), ((), ())),
                         preferred_element_type=jnp.float32)
    xx_t = jnp.sum(a * a, axis=1, keepdims=True)
    xx_r = jnp.sum(xn * xn, axis=0, keepdims=True)
    d_ref[...] = 2.0 * ip - xx_t - xx_r

    iota = lax.broadcasted_iota(jnp.int32, (a.shape[0], xn.shape[1]), 1)
    kiota = lax.broadcasted_iota(jnp.int32, (a.shape[0], K), 1)
    iscr_ref[...] = jnp.zeros((a.shape[0], K), jnp.int32)

    def body(k, rm):
        d = d_ref[...]
        am = jnp.min(jnp.where(d >= rm, iota, jnp.int32(N)), axis=1,
                     keepdims=True)
        iscr_ref[...] = iscr_ref[...] + jnp.where(kiota == k, am, 0)
        sel = (iota == am)
        dn = jnp.where(sel, NEG_INF, d)
        d_ref[...] = dn
        return jnp.max(dn, axis=1, keepdims=True)

    rm0 = jnp.max(d_ref[...], axis=1, keepdims=True)
    lax.fori_loop(0, K, body, rm0)
    idx_ref[0] = iscr_ref[...] + pl.program_id(0) * N


def _knn(xt, xn):
    B, n, C = xt.shape
    return pl.pallas_call(
        _knn_body,
        grid=(B, n // TN),
        in_specs=[
            pl.BlockSpec((1, TN, C), lambda b, t: (b, t, 0)),
            pl.BlockSpec((1, C, n), lambda b, t: (b, 0, 0)),
        ],
        out_specs=pl.BlockSpec((1, TN, K), lambda b, t: (b, t, 0)),
        out_shape=jax.ShapeDtypeStruct((B, n, K), jnp.int32),
        scratch_shapes=[pltpu.VMEM((TN, n), jnp.float32),
                        pltpu.VMEM((TN, K), jnp.int32)],
    )(xt, xn)


def _sc_gather(table, idxflat):
    M = idxflat.shape[0]
    C = table.shape[1]
    W = 128
    idx2 = idxflat.reshape(1, M)
    mesh = plsc.VectorSubcoreMesh(core_axis_name="core",
                                  subcore_axis_name="subcore")

    @functools.partial(
        pl.kernel,
        out_type=jax.ShapeDtypeStruct((M, C), jnp.float32),
        mesh=mesh,
        compiler_params=pltpu.CompilerParams(use_tc_tiling_on_sc=False),
    )
    def kern(x_hbm, i_hbm, o_hbm):
        def body(i_vmem, o_vmem):
            pltpu.sync_copy(x_hbm.at[i_vmem.at[0]], o_vmem)

        pltpu.emit_pipeline(
            body,
            grid=(M // W,),
            in_specs=[pl.BlockSpec((1, W), index_map=lambda i: (0, i))],
            out_specs=[pl.BlockSpec((W, C), index_map=lambda i: (i, 0))],
            core_axis_name=("core", "subcore"),
            dimension_semantics=(pltpu.PARALLEL,),
        )(i_hbm, o_hbm)

    return kern(table, idx2)


def _econv_body(g_ref, xt_ref, w_ref, mx_ref, mn_ref, sp_ref, qp_ref):
    g2 = g_ref[0]
    xb = xt_ref[0]
    C = xb.shape[1]
    xbro = jnp.broadcast_to(xb[:, None, :], (TNB, K, C)).reshape(TNB * K, C)
    fcat = jnp.concatenate([g2 - xbro, xbro], axis=1)
    t = jnp.dot(fcat, w_ref[...], preferred_element_type=jnp.float32)
    O = t.shape[1]
    t3 = t.reshape(TNB, K, O)
    mx_ref[0] = jnp.max(t3, axis=1)
    mn_ref[0] = jnp.min(t3, axis=1)
    sp_ref[0, 0] = jnp.sum(t, axis=0, keepdims=True)
    qp_ref[0, 0] = jnp.sum(t * t, axis=0, keepdims=True)


def _econv(g, xt, w2T):
    B, n, C = xt.shape
    O = w2T.shape[1]
    NT = n // TNB
    return pl.pallas_call(
        _econv_body,
        grid=(B, NT),
        in_specs=[
            pl.BlockSpec((1, TNB * K, C), lambda b, t: (b, t, 0)),
            pl.BlockSpec((1, TNB, C), lambda b, t: (b, t, 0)),
            pl.BlockSpec((2 * C, O), lambda b, t: (0, 0)),
        ],
        out_specs=[
            pl.BlockSpec((1, TNB, O), lambda b, t: (b, t, 0)),
            pl.BlockSpec((1, TNB, O), lambda b, t: (b, t, 0)),
            pl.BlockSpec((1, 1, 1, O), lambda b, t: (b, t, 0, 0)),
            pl.BlockSpec((1, 1, 1, O), lambda b, t: (b, t, 0, 0)),
        ],
        out_shape=[jax.ShapeDtypeStruct((B, n, O), jnp.float32),
                   jax.ShapeDtypeStruct((B, n, O), jnp.float32),
                   jax.ShapeDtypeStruct((B, NT, 1, O), jnp.float32),
                   jax.ShapeDtypeStruct((B, NT, 1, O), jnp.float32)],
    )(g, xt, w2T)


def _group_norm_consts(s, q, cnt, G):
    O = s.shape[1]
    W = O // G
    mc, vc = [], []
    for g in range(G):
        sg = jnp.sum(s[:, g * W:(g + 1) * W], axis=1, keepdims=True) / cnt
        qg = jnp.sum(q[:, g * W:(g + 1) * W], axis=1, keepdims=True) / cnt
        mc.append(jnp.broadcast_to(sg, (1, W)))
        vc.append(jnp.broadcast_to(qg - sg * sg, (1, W)))
    return jnp.concatenate(mc, axis=1), jnp.concatenate(vc, axis=1)


def _fin_body(mx_ref, mn_ref, sp_ref, qp_ref, g_ref, b_ref,
              xt_ref, xn_ref, *, cnt, slope, groups):
    s = jnp.sum(sp_ref[0, :, 0, :], axis=0, keepdims=True)
    q = jnp.sum(qp_ref[0, :, 0, :], axis=0, keepdims=True)
    mean_c, var_c = _group_norm_consts(s, q, cnt, groups)
    den = jnp.sqrt(var_c + G_EPS)
    gam = g_ref[...]
    bet = b_ref[...]
    sel_v = jnp.where(gam >= 0, mx_ref[0], mn_ref[0])
    xh = (sel_v - mean_c) / den * gam + bet
    xo = jnp.where(xh >= 0, xh, slope * xh)
    xt_ref[0] = xo
    xn_ref[0] = xo.T


def _finalize(mx, mn, sp, qp, gam, bet, groups, slope):
    B, n, O = mx.shape
    cnt = float((O // groups) * n * K)
    body = functools.partial(_fin_body, cnt=cnt, slope=slope, groups=groups)
    NT = sp.shape[1]
    assert sp.shape == (B, NT, 1, O)
    return pl.pallas_call(
        body,
        grid=(B,),
        in_specs=[
            pl.BlockSpec((1, n, O), lambda b: (b, 0, 0)),
            pl.BlockSpec((1, n, O), lambda b: (b, 0, 0)),
            pl.BlockSpec((1, NT, 1, O), lambda b: (b, 0, 0, 0)),
            pl.BlockSpec((1, NT, 1, O), lambda b: (b, 0, 0, 0)),
            pl.BlockSpec((1, O), lambda b: (0, 0)),
            pl.BlockSpec((1, O), lambda b: (0, 0)),
        ],
        out_specs=[
            pl.BlockSpec((1, n, O), lambda b: (b, 0, 0)),
            pl.BlockSpec((1, O, n), lambda b: (b, 0, 0)),
        ],
        out_shape=[jax.ShapeDtypeStruct((B, n, O), jnp.float32),
                   jax.ShapeDtypeStruct((B, O, n), jnp.float32)],
    )(mx, mn, sp, qp, gam.reshape(1, O), bet.reshape(1, O))


def _edge_layer(xt, xn, w2T, gam, bet, groups=2, slope=0.2):
    B, n, C = xt.shape
    idx = _knn(xt, xn)
    g = _sc_gather(xt.reshape(B * n, C), idx.reshape(B * n * K))
    g = g.reshape(B, n * K, C)
    mx, mn, sp, qp = _econv(g, xt, w2T)
    return _finalize(mx, mn, sp, qp, gam, bet, groups, slope)


def _gn_rows(t, gam, bet, groups, cnt):
    col_s = jnp.sum(t, axis=0, keepdims=True)
    col_q = jnp.sum(t * t, axis=0, keepdims=True)
    mean_c, var_c = _group_norm_consts(col_s, col_q, cnt, groups)
    return (t - mean_c) / jnp.sqrt(var_c + G_EPS) * gam + bet


def _head_body(x1_ref, x2_ref, x3_ref,
               mlp1_ref, mlp1b_ref, gnm_g_ref, gnm_b_ref,
               c1_ref, c1b_ref, bn1_g_ref, bn1_b_ref,
               c2_ref, c2b_ref, bn2_g_ref, bn2_b_ref,
               s1_ref, s1b_ref, bns_g_ref, bns_b_ref,
               s2_ref, s2b_ref,
               p1_ref, p1b_ref, bnp_g_ref, bnp_b_ref,
               p2_ref, p2b_ref,
               emb_ref, lp_ref):
    n = x1_ref.shape[1]
    xf = jnp.concatenate([x1_ref[0], x2_ref[0], x3_ref[0]], axis=1)
    h = jnp.dot(xf, mlp1_ref[...], preferred_element_type=jnp.float32) \
        + mlp1b_ref[...]
    h = _gn_rows(h, gnm_g_ref[...], gnm_b_ref[...], 8, float(n * 128))
    h = jnp.maximum(h, 0.0)
    x4 = jnp.max(h, axis=0, keepdims=True)
    hh = jnp.concatenate([jnp.broadcast_to(x4, (n, 1024)), xf], axis=1)
    h1 = jnp.dot(hh, c1_ref[...], preferred_element_type=jnp.float32) \
        + c1b_ref[...]
    h1 = _gn_rows(h1, bn1_g_ref[...], bn1_b_ref[...], 8, float(n * 64))
    h1 = jnp.maximum(h1, 0.0)
    xa = jnp.dot(h1, c2_ref[...], preferred_element_type=jnp.float32) \
        + c2b_ref[...]
    xa = _gn_rows(xa, bn2_g_ref[...], bn2_b_ref[...], 4, float(n * 64))
    xa = jnp.maximum(xa, 0.0)
    e = jnp.dot(xa, s1_ref[...], preferred_element_type=jnp.float32) \
        + s1b_ref[...]
    e = _gn_rows(e, bns_g_ref[...], bns_b_ref[...], 4, float(n * 64))
    e = jnp.maximum(e, 0.0)
    emb_ref[0] = jnp.dot(e, s2_ref[...], preferred_element_type=jnp.float32) \
        + s2b_ref[...]
    q = jnp.dot(xa, p1_ref[...], preferred_element_type=jnp.float32) \
        + p1b_ref[...]
    q = _gn_rows(q, bnp_g_ref[...], bnp_b_ref[...], 4, float(n * 64))
    q = jnp.maximum(q, 0.0)
    logits = jnp.dot(q, p2_ref[...], preferred_element_type=jnp.float32) \
        + p2b_ref[...]
    lm = jnp.max(logits, axis=1, keepdims=True)
    shifted = logits - lm
    lp_ref[0] = shifted - jnp.log(jnp.sum(jnp.exp(shifted), axis=1,
                                          keepdims=True))


def _head(x1t, x2t, x3t, p):
    B, n, _ = x1t.shape
    r = lambda v: v.reshape(1, -1)
    const = lambda shp: pl.BlockSpec(shp, lambda b: tuple(0 for _ in shp))
    args = [
        p['mlp1_w'].T, r(p['mlp1_b']), r(p['gnm_g']), r(p['gnm_b']),
        p['c1_w'].T, r(p['c1_b']), r(p['bn1_g']), r(p['bn1_b']),
        p['c2_w'].T, r(p['c2_b']), r(p['bn2_g']), r(p['bn2_b']),
        p['seg1_w'].T, r(p['seg1_b']), r(p['bnseg_g']), r(p['bnseg_b']),
        p['seg2_w'].T, r(p['seg2_b']),
        p['prim1_w'].T, r(p['prim1_b']), r(p['bnprim_g']), r(p['bnprim_b']),
        p['prim2_w'].T, r(p['prim2_b']),
    ]
    in_specs = [
        pl.BlockSpec((1, n, 64), lambda b: (b, 0, 0)),
        pl.BlockSpec((1, n, 64), lambda b: (b, 0, 0)),
        pl.BlockSpec((1, n, 128), lambda b: (b, 0, 0)),
    ] + [const(a.shape) for a in args]
    return pl.pallas_call(
        _head_body,
        grid=(B,),
        in_specs=in_specs,
        out_specs=[
            pl.BlockSpec((1, n, 50), lambda b: (b, 0, 0)),
            pl.BlockSpec((1, n, 8), lambda b: (b, 0, 0)),
        ],
        out_shape=[jax.ShapeDtypeStruct((B, n, 50), jnp.float32),
                   jax.ShapeDtypeStruct((B, n, 8), jnp.float32)],
    )(x1t, x2t, x3t, *args)


def kernel(x, conv1_w, gn1_g, gn1_b, conv2_w, gn2_g, gn2_b, conv3_w, gn3_g,
           gn3_b, mlp1_w, mlp1_b, gnm_g, gnm_b, c1_w, c1_b, bn1_g, bn1_b,
           c2_w, c2_b, bn2_g, bn2_b, seg1_w, seg1_b, bnseg_g, bnseg_b,
           seg2_w, seg2_b, prim1_w, prim1_b, bnprim_g, bnprim_b,
           prim2_w, prim2_b):
    B, C0, n = x.shape
    CP = 16
    xn0 = jnp.concatenate([x, jnp.zeros((B, CP - C0, n), jnp.float32)], axis=1)
    xt0 = jnp.transpose(xn0, (0, 2, 1))
    zpad = jnp.zeros((64, CP - C0), jnp.float32)
    w1 = jnp.concatenate([conv1_w[:, :C0], zpad,
                          conv1_w[:, C0:], zpad], axis=1).T
    x1t, x1n = _edge_layer(xt0, xn0, w1, gn1_g, gn1_b)
    x2t, x2n = _edge_layer(x1t, x1n, conv2_w.T, gn2_g, gn2_b)
    x3t, _ = _edge_layer(x2t, x2n, conv3_w.T, gn3_g, gn3_b)
    p = dict(mlp1_w=mlp1_w, mlp1_b=mlp1_b, gnm_g=gnm_g, gnm_b=gnm_b,
             c1_w=c1_w, c1_b=c1_b, bn1_g=bn1_g, bn1_b=bn1_b,
             c2_w=c2_w, c2_b=c2_b, bn2_g=bn2_g, bn2_b=bn2_b,
             seg1_w=seg1_w, seg1_b=seg1_b, bnseg_g=bnseg_g, bnseg_b=bnseg_b,
             seg2_w=seg2_w, seg2_b=seg2_b,
             prim1_w=prim1_w, prim1_b=prim1_b, bnprim_g=bnprim_g,
             bnprim_b=bnprim_b, prim2_w=prim2_w, prim2_b=prim2_b)
    emb, lp = _head(x1t, x2t, x3t, p)
    return (jnp.transpose(emb, (0, 2, 1)), jnp.transpose(lp, (0, 2, 1)))


# speedup vs baseline: 1.3084x; 1.0103x over previous
import functools
import numpy as np
import jax
import jax.numpy as jnp
from jax import lax
from jax.experimental import pallas as pl
from jax.experimental.pallas import tpu as pltpu
from jax.experimental.pallas import tpu_sc as plsc

K = 80
G_EPS = 1e-5
N = 2048
TN = 2048
TNB = 128
NEG_INF = float('-inf')


def _knn_body(xt_ref, xn_ref, idx_ref, d_ref, iscr_ref):
    a = xt_ref[0]
    xn = xn_ref[0]
    t = pl.program_id(1)
    a_cn = xn_ref[0, :, pl.ds(t * TN, TN)]
    ip = lax.dot_general(a_cn, xn, (((0,), (0,)), ((), ())),
                         preferred_element_type=jnp.float32)
    xx_t = jnp.sum(a * a, axis=1, keepdims=True)
    xx_r = jnp.sum(xn * xn, axis=0, keepdims=True)
    d_ref[...] = 2.0 * ip - xx_t - xx_r

    iota = lax.broadcasted_iota(jnp.int32, (a.shape[0], xn.shape[1]), 1)
    kiota = lax.broadcasted_iota(jnp.int32, (a.shape[0], K), 1)
    iscr_ref[...] = jnp.zeros((a.shape[0], K), jnp.int32)

    def body(k, rm):
        d = d_ref[...]
        am = jnp.min(jnp.where(d >= rm, iota, jnp.int32(N)), axis=1,
                     keepdims=True)
        iscr_ref[...] = iscr_ref[...] + jnp.where(kiota == k, am, 0)
        sel = (iota == am)
        dn = jnp.where(sel, NEG_INF, d)
        d_ref[...] = dn
        return jnp.max(dn, axis=1, keepdims=True)

    rm0 = jnp.max(d_ref[...], axis=1, keepdims=True)
    lax.fori_loop(0, K, body, rm0)
    idx_ref[0] = iscr_ref[...] + pl.program_id(0) * N


def _knn(xt, xn):
    B, n, C = xt.shape
    return pl.pallas_call(
        _knn_body,
        grid=(B, n // TN),
        in_specs=[
            pl.BlockSpec((1, TN, C), lambda b, t: (b, t, 0)),
            pl.BlockSpec((1, C, n), lambda b, t: (b, 0, 0)),
        ],
        out_specs=pl.BlockSpec((1, TN, K), lambda b, t: (b, t, 0)),
        out_shape=jax.ShapeDtypeStruct((B, n, K), jnp.int32),
        scratch_shapes=[pltpu.VMEM((TN, n), jnp.float32),
                        pltpu.VMEM((TN, K), jnp.int32)],
    )(xt, xn)


def _sc_gather(table, idxflat):
    M = idxflat.shape[0]
    C = table.shape[1]
    W = 128
    idx2 = idxflat.reshape(1, M)
    mesh = plsc.VectorSubcoreMesh(core_axis_name="core",
                                  subcore_axis_name="subcore")

    @functools.partial(
        pl.kernel,
        out_type=jax.ShapeDtypeStruct((M, C), jnp.float32),
        mesh=mesh,
        compiler_params=pltpu.CompilerParams(use_tc_tiling_on_sc=False),
    )
    def kern(x_hbm, i_hbm, o_hbm):
        def body(i_vmem, o_vmem):
            pltpu.sync_copy(x_hbm.at[i_vmem.at[0]], o_vmem)

        pltpu.emit_pipeline(
            body,
            grid=(M // W,),
            in_specs=[pl.BlockSpec((1, W), index_map=lambda i: (0, i))],
            out_specs=[pl.BlockSpec((W, C), index_map=lambda i: (i, 0))],
            core_axis_name=("core", "subcore"),
            dimension_semantics=(pltpu.PARALLEL,),
        )(i_hbm, o_hbm)

    return kern(table, idx2)


def _econv_body(g_ref, xt_ref, w_ref, mx_ref, mn_ref, sp_ref, qp_ref):
    g2 = g_ref[0]
    xb = xt_ref[0]
    C = xb.shape[1]
    xbro = jnp.broadcast_to(xb[:, None, :], (TNB, K, C)).reshape(TNB * K, C)
    fcat = jnp.concatenate([g2 - xbro, xbro], axis=1)
    t = jnp.dot(fcat, w_ref[...], preferred_element_type=jnp.float32)
    O = t.shape[1]
    t3 = t.reshape(TNB, K, O)
    mx_ref[0] = jnp.max(t3, axis=1)
    mn_ref[0] = jnp.min(t3, axis=1)
    sp_ref[0, 0] = jnp.sum(t, axis=0, keepdims=True)
    qp_ref[0, 0] = jnp.sum(t * t, axis=0, keepdims=True)


def _econv(g, xt, w2T):
    B, n, C = xt.shape
    O = w2T.shape[1]
    NT = n // TNB
    return pl.pallas_call(
        _econv_body,
        grid=(B, NT),
        in_specs=[
            pl.BlockSpec((1, TNB * K, C), lambda b, t: (b, t, 0)),
            pl.BlockSpec((1, TNB, C), lambda b, t: (b, t, 0)),
            pl.BlockSpec((2 * C, O), lambda b, t: (0, 0)),
        ],
        out_specs=[
            pl.BlockSpec((1, TNB, O), lambda b, t: (b, t, 0)),
            pl.BlockSpec((1, TNB, O), lambda b, t: (b, t, 0)),
            pl.BlockSpec((1, 1, 1, O), lambda b, t: (b, t, 0, 0)),
            pl.BlockSpec((1, 1, 1, O), lambda b, t: (b, t, 0, 0)),
        ],
        out_shape=[jax.ShapeDtypeStruct((B, n, O), jnp.float32),
                   jax.ShapeDtypeStruct((B, n, O), jnp.float32),
                   jax.ShapeDtypeStruct((B, NT, 1, O), jnp.float32),
                   jax.ShapeDtypeStruct((B, NT, 1, O), jnp.float32)],
    )(g, xt, w2T)


def _group_norm_consts(s, q, cnt, G):
    O = s.shape[1]
    W = O // G
    mc, vc = [], []
    for g in range(G):
        sg = jnp.sum(s[:, g * W:(g + 1) * W], axis=1, keepdims=True) / cnt
        qg = jnp.sum(q[:, g * W:(g + 1) * W], axis=1, keepdims=True) / cnt
        mc.append(jnp.broadcast_to(sg, (1, W)))
        vc.append(jnp.broadcast_to(qg - sg * sg, (1, W)))
    return jnp.concatenate(mc, axis=1), jnp.concatenate(vc, axis=1)


def _fin_body(mx_ref, mn_ref, sp_ref, qp_ref, g_ref, b_ref,
              xt_ref, xn_ref, *, cnt, slope, groups):
    s = jnp.sum(sp_ref[0, :, 0, :], axis=0, keepdims=True)
    q = jnp.sum(qp_ref[0, :, 0, :], axis=0, keepdims=True)
    mean_c, var_c = _group_norm_consts(s, q, cnt, groups)
    den = jnp.sqrt(var_c + G_EPS)
    gam = g_ref[...]
    bet = b_ref[...]
    sel_v = jnp.where(gam >= 0, mx_ref[0], mn_ref[0])
    xh = (sel_v - mean_c) / den * gam + bet
    xo = jnp.where(xh >= 0, xh, slope * xh)
    xt_ref[0] = xo
    xn_ref[0] = xo.T


def _finalize(mx, mn, sp, qp, gam, bet, groups, slope):
    B, n, O = mx.shape
    cnt = float((O // groups) * n * K)
    body = functools.partial(_fin_body, cnt=cnt, slope=slope, groups=groups)
    NT = sp.shape[1]
    assert sp.shape == (B, NT, 1, O)
    return pl.pallas_call(
        body,
        grid=(B,),
        in_specs=[
            pl.BlockSpec((1, n, O), lambda b: (b, 0, 0)),
            pl.BlockSpec((1, n, O), lambda b: (b, 0, 0)),
            pl.BlockSpec((1, NT, 1, O), lambda b: (b, 0, 0, 0)),
            pl.BlockSpec((1, NT, 1, O), lambda b: (b, 0, 0, 0)),
            pl.BlockSpec((1, O), lambda b: (0, 0)),
            pl.BlockSpec((1, O), lambda b: (0, 0)),
        ],
        out_specs=[
            pl.BlockSpec((1, n, O), lambda b: (b, 0, 0)),
            pl.BlockSpec((1, O, n), lambda b: (b, 0, 0)),
        ],
        out_shape=[jax.ShapeDtypeStruct((B, n, O), jnp.float32),
                   jax.ShapeDtypeStruct((B, O, n), jnp.float32)],
    )(mx, mn, sp, qp, gam.reshape(1, O), bet.reshape(1, O))


def _edge_layer(xt, xn, w2T, gam, bet, groups=2, slope=0.2):
    B, n, C = xt.shape
    idx = _knn(xt, xn)
    g = _sc_gather(xt.reshape(B * n, C), idx.reshape(B * n * K))
    g = g.reshape(B, n * K, C)
    mx, mn, sp, qp = _econv(g, xt, w2T)
    return _finalize(mx, mn, sp, qp, gam, bet, groups, slope)


def _gn_rows(t, gam, bet, groups, cnt):
    col_s = jnp.sum(t, axis=0, keepdims=True)
    col_q = jnp.sum(t * t, axis=0, keepdims=True)
    mean_c, var_c = _group_norm_consts(col_s, col_q, cnt, groups)
    return (t - mean_c) / jnp.sqrt(var_c + G_EPS) * gam + bet


def _head_body(x1_ref, x2_ref, x3_ref,
               mlp1_ref, mlp1b_ref, gnm_g_ref, gnm_b_ref,
               c1_ref, c1b_ref, bn1_g_ref, bn1_b_ref,
               c2_ref, c2b_ref, bn2_g_ref, bn2_b_ref,
               s1_ref, s1b_ref, bns_g_ref, bns_b_ref,
               s2_ref, s2b_ref,
               p1_ref, p1b_ref, bnp_g_ref, bnp_b_ref,
               p2_ref, p2b_ref,
               emb_ref, lp_ref):
    n = x1_ref.shape[1]
    xf = jnp.concatenate([x1_ref[0], x2_ref[0], x3_ref[0]], axis=1)
    h = jnp.dot(xf, mlp1_ref[...], preferred_element_type=jnp.float32) \
        + mlp1b_ref[...]
    h = _gn_rows(h, gnm_g_ref[...], gnm_b_ref[...], 8, float(n * 128))
    h = jnp.maximum(h, 0.0)
    x4 = jnp.max(h, axis=0, keepdims=True)
    hh = jnp.concatenate([jnp.broadcast_to(x4, (n, 1024)), xf], axis=1)
    h1 = jnp.dot(hh, c1_ref[...], preferred_element_type=jnp.float32) \
        + c1b_ref[...]
    h1 = _gn_rows(h1, bn1_g_ref[...], bn1_b_ref[...], 8, float(n * 64))
    h1 = jnp.maximum(h1, 0.0)
    xa = jnp.dot(h1, c2_ref[...], preferred_element_type=jnp.float32) \
        + c2b_ref[...]
    xa = _gn_rows(xa, bn2_g_ref[...], bn2_b_ref[...], 4, float(n * 64))
    xa = jnp.maximum(xa, 0.0)
    e = jnp.dot(xa, s1_ref[...], preferred_element_type=jnp.float32) \
        + s1b_ref[...]
    e = _gn_rows(e, bns_g_ref[...], bns_b_ref[...], 4, float(n * 64))
    e = jnp.maximum(e, 0.0)
    emb_ref[0] = jnp.dot(e, s2_ref[...], preferred_element_type=jnp.float32) \
        + s2b_ref[...]
    q = jnp.dot(xa, p1_ref[...], preferred_element_type=jnp.float32) \
        + p1b_ref[...]
    q = _gn_rows(q, bnp_g_ref[...], bnp_b_ref[...], 4, float(n * 64))
    q = jnp.maximum(q, 0.0)
    logits = jnp.dot(q, p2_ref[...], preferred_element_type=jnp.float32) \
        + p2b_ref[...]
    lm = jnp.max(logits, axis=1, keepdims=True)
    shifted = logits - lm
    lp_ref[0] = shifted - jnp.log(jnp.sum(jnp.exp(shifted), axis=1,
                                          keepdims=True))


def _head(x1t, x2t, x3t, p):
    B, n, _ = x1t.shape
    r = lambda v: v.reshape(1, -1)
    const = lambda shp: pl.BlockSpec(shp, lambda b: tuple(0 for _ in shp))
    args = [
        p['mlp1_w'].T, r(p['mlp1_b']), r(p['gnm_g']), r(p['gnm_b']),
        p['c1_w'].T, r(p['c1_b']), r(p['bn1_g']), r(p['bn1_b']),
        p['c2_w'].T, r(p['c2_b']), r(p['bn2_g']), r(p['bn2_b']),
        p['seg1_w'].T, r(p['seg1_b']), r(p['bnseg_g']), r(p['bnseg_b']),
        p['seg2_w'].T, r(p['seg2_b']),
        p['prim1_w'].T, r(p['prim1_b']), r(p['bnprim_g']), r(p['bnprim_b']),
        p['prim2_w'].T, r(p['prim2_b']),
    ]
    in_specs = [
        pl.BlockSpec((1, n, 64), lambda b: (b, 0, 0)),
        pl.BlockSpec((1, n, 64), lambda b: (b, 0, 0)),
        pl.BlockSpec((1, n, 128), lambda b: (b, 0, 0)),
    ] + [const(a.shape) for a in args]
    return pl.pallas_call(
        _head_body,
        grid=(B,),
        in_specs=in_specs,
        out_specs=[
            pl.BlockSpec((1, n, 50), lambda b: (b, 0, 0)),
            pl.BlockSpec((1, n, 8), lambda b: (b, 0, 0)),
        ],
        out_shape=[jax.ShapeDtypeStruct((B, n, 50), jnp.float32),
                   jax.ShapeDtypeStruct((B, n, 8), jnp.float32)],
    )(x1t, x2t, x3t, *args)


def kernel(x, conv1_w, gn1_g, gn1_b, conv2_w, gn2_g, gn2_b, conv3_w, gn3_g,
           gn3_b, mlp1_w, mlp1_b, gnm_g, gnm_b, c1_w, c1_b, bn1_g, bn1_b,
           c2_w, c2_b, bn2_g, bn2_b, seg1_w, seg1_b, bnseg_g, bnseg_b,
           seg2_w, seg2_b, prim1_w, prim1_b, bnprim_g, bnprim_b,
           prim2_w, prim2_b):
    B, C0, n = x.shape
    CP = 16
    xn0 = jnp.concatenate([x, jnp.zeros((B, CP - C0, n), jnp.float32)], axis=1)
    xt0 = jnp.transpose(xn0, (0, 2, 1))
    zpad = jnp.zeros((64, CP - C0), jnp.float32)
    w1 = jnp.concatenate([conv1_w[:, :C0], zpad,
                          conv1_w[:, C0:], zpad], axis=1).T
    x1t, x1n = _edge_layer(xt0, xn0, w1, gn1_g, gn1_b)
    x2t, x2n = _edge_layer(x1t, x1n, conv2_w.T, gn2_g, gn2_b)
    x3t, _ = _edge_layer(x2t, x2n, conv3_w.T, gn3_g, gn3_b)
    p = dict(mlp1_w=mlp1_w, mlp1_b=mlp1_b, gnm_g=gnm_g, gnm_b=gnm_b,
             c1_w=c1_w, c1_b=c1_b, bn1_g=bn1_g, bn1_b=bn1_b,
             c2_w=c2_w, c2_b=c2_b, bn2_g=bn2_g, bn2_b=bn2_b,
             seg1_w=seg1_w, seg1_b=seg1_b, bnseg_g=bnseg_g, bnseg_b=bnseg_b,
             seg2_w=seg2_w, seg2_b=seg2_b,
             prim1_w=prim1_w, prim1_b=prim1_b, bnprim_g=bnprim_g,
             bnprim_b=bnprim_b, prim2_w=prim2_w, prim2_b=prim2_b)
    emb, lp = _head(x1t, x2t, x3t, p)
    return (jnp.transpose(emb, (0, 2, 1)), jnp.transpose(lp, (0, 2, 1)))


# speedup vs baseline: 1.3173x; 1.0067x over previous
import functools
import numpy as np
import jax
import jax.numpy as jnp
from jax import lax
from jax.experimental import pallas as pl
from jax.experimental.pallas import tpu as pltpu
from jax.experimental.pallas import tpu_sc as plsc

K = 80
G_EPS = 1e-5
N = 2048
TN = 2048
TNB = 256
NEG_INF = float('-inf')


def _knn_body(xt_ref, xn_ref, idx_ref, d_ref, iscr_ref):
    a = xt_ref[0]
    xn = xn_ref[0]
    t = pl.program_id(1)
    a_cn = xn_ref[0, :, pl.ds(t * TN, TN)]
    ip = lax.dot_general(a_cn, xn, (((0,), (0,)), ((), ())),
                         preferred_element_type=jnp.float32)
    xx_t = jnp.sum(a * a, axis=1, keepdims=True)
    xx_r = jnp.sum(xn * xn, axis=0, keepdims=True)
    d_ref[...] = 2.0 * ip - xx_t - xx_r

    iota = lax.broadcasted_iota(jnp.int32, (a.shape[0], xn.shape[1]), 1)
    kiota = lax.broadcasted_iota(jnp.int32, (a.shape[0], K), 1)
    iscr_ref[...] = jnp.zeros((a.shape[0], K), jnp.int32)

    def body(k, rm):
        d = d_ref[...]
        am = jnp.min(jnp.where(d >= rm, iota, jnp.int32(N)), axis=1,
                     keepdims=True)
        iscr_ref[...] = iscr_ref[...] + jnp.where(kiota == k, am, 0)
        sel = (iota == am)
        dn = jnp.where(sel, NEG_INF, d)
        d_ref[...] = dn
        return jnp.max(dn, axis=1, keepdims=True)

    rm0 = jnp.max(d_ref[...], axis=1, keepdims=True)
    lax.fori_loop(0, K, body, rm0)
    idx_ref[0] = iscr_ref[...] + pl.program_id(0) * N


def _knn(xt, xn):
    B, n, C = xt.shape
    return pl.pallas_call(
        _knn_body,
        grid=(B, n // TN),
        in_specs=[
            pl.BlockSpec((1, TN, C), lambda b, t: (b, t, 0)),
            pl.BlockSpec((1, C, n), lambda b, t: (b, 0, 0)),
        ],
        out_specs=pl.BlockSpec((1, TN, K), lambda b, t: (b, t, 0)),
        out_shape=jax.ShapeDtypeStruct((B, n, K), jnp.int32),
        scratch_shapes=[pltpu.VMEM((TN, n), jnp.float32),
                        pltpu.VMEM((TN, K), jnp.int32)],
    )(xt, xn)


def _sc_gather(table, idxflat):
    M = idxflat.shape[0]
    C = table.shape[1]
    W = 128
    idx2 = idxflat.reshape(1, M)
    mesh = plsc.VectorSubcoreMesh(core_axis_name="core",
                                  subcore_axis_name="subcore")

    @functools.partial(
        pl.kernel,
        out_type=jax.ShapeDtypeStruct((M, C), jnp.float32),
        mesh=mesh,
        compiler_params=pltpu.CompilerParams(use_tc_tiling_on_sc=False),
    )
    def kern(x_hbm, i_hbm, o_hbm):
        def body(i_vmem, o_vmem):
            pltpu.sync_copy(x_hbm.at[i_vmem.at[0]], o_vmem)

        pltpu.emit_pipeline(
            body,
            grid=(M // W,),
            in_specs=[pl.BlockSpec((1, W), index_map=lambda i: (0, i))],
            out_specs=[pl.BlockSpec((W, C), index_map=lambda i: (i, 0))],
            core_axis_name=("core", "subcore"),
            dimension_semantics=(pltpu.PARALLEL,),
        )(i_hbm, o_hbm)

    return kern(table, idx2)


def _econv_body(g_ref, xt_ref, w_ref, mx_ref, mn_ref, sp_ref, qp_ref):
    g2 = g_ref[0]
    xb = xt_ref[0]
    C = xb.shape[1]
    xbro = jnp.broadcast_to(xb[:, None, :], (TNB, K, C)).reshape(TNB * K, C)
    fcat = jnp.concatenate([g2 - xbro, xbro], axis=1)
    t = jnp.dot(fcat, w_ref[...], preferred_element_type=jnp.float32)
    O = t.shape[1]
    t3 = t.reshape(TNB, K, O)
    mx_ref[0] = jnp.max(t3, axis=1)
    mn_ref[0] = jnp.min(t3, axis=1)
    sp_ref[0, 0] = jnp.sum(t, axis=0, keepdims=True)
    qp_ref[0, 0] = jnp.sum(t * t, axis=0, keepdims=True)


def _econv(g, xt, w2T):
    B, n, C = xt.shape
    O = w2T.shape[1]
    NT = n // TNB
    return pl.pallas_call(
        _econv_body,
        grid=(B, NT),
        in_specs=[
            pl.BlockSpec((1, TNB * K, C), lambda b, t: (b, t, 0)),
            pl.BlockSpec((1, TNB, C), lambda b, t: (b, t, 0)),
            pl.BlockSpec((2 * C, O), lambda b, t: (0, 0)),
        ],
        out_specs=[
            pl.BlockSpec((1, TNB, O), lambda b, t: (b, t, 0)),
            pl.BlockSpec((1, TNB, O), lambda b, t: (b, t, 0)),
            pl.BlockSpec((1, 1, 1, O), lambda b, t: (b, t, 0, 0)),
            pl.BlockSpec((1, 1, 1, O), lambda b, t: (b, t, 0, 0)),
        ],
        out_shape=[jax.ShapeDtypeStruct((B, n, O), jnp.float32),
                   jax.ShapeDtypeStruct((B, n, O), jnp.float32),
                   jax.ShapeDtypeStruct((B, NT, 1, O), jnp.float32),
                   jax.ShapeDtypeStruct((B, NT, 1, O), jnp.float32)],
    )(g, xt, w2T)


def _group_norm_consts(s, q, cnt, G):
    O = s.shape[1]
    W = O // G
    mc, vc = [], []
    for g in range(G):
        sg = jnp.sum(s[:, g * W:(g + 1) * W], axis=1, keepdims=True) / cnt
        qg = jnp.sum(q[:, g * W:(g + 1) * W], axis=1, keepdims=True) / cnt
        mc.append(jnp.broadcast_to(sg, (1, W)))
        vc.append(jnp.broadcast_to(qg - sg * sg, (1, W)))
    return jnp.concatenate(mc, axis=1), jnp.concatenate(vc, axis=1)


def _fin_body(mx_ref, mn_ref, sp_ref, qp_ref, g_ref, b_ref,
              xt_ref, xn_ref, *, cnt, slope, groups):
    s = jnp.sum(sp_ref[0, :, 0, :], axis=0, keepdims=True)
    q = jnp.sum(qp_ref[0, :, 0, :], axis=0, keepdims=True)
    mean_c, var_c = _group_norm_consts(s, q, cnt, groups)
    den = jnp.sqrt(var_c + G_EPS)
    gam = g_ref[...]
    bet = b_ref[...]
    sel_v = jnp.where(gam >= 0, mx_ref[0], mn_ref[0])
    xh = (sel_v - mean_c) / den * gam + bet
    xo = jnp.where(xh >= 0, xh, slope * xh)
    xt_ref[0] = xo
    xn_ref[0] = xo.T


def _finalize(mx, mn, sp, qp, gam, bet, groups, slope):
    B, n, O = mx.shape
    cnt = float((O // groups) * n * K)
    body = functools.partial(_fin_body, cnt=cnt, slope=slope, groups=groups)
    NT = sp.shape[1]
    assert sp.shape == (B, NT, 1, O)
    return pl.pallas_call(
        body,
        grid=(B,),
        in_specs=[
            pl.BlockSpec((1, n, O), lambda b: (b, 0, 0)),
            pl.BlockSpec((1, n, O), lambda b: (b, 0, 0)),
            pl.BlockSpec((1, NT, 1, O), lambda b: (b, 0, 0, 0)),
            pl.BlockSpec((1, NT, 1, O), lambda b: (b, 0, 0, 0)),
            pl.BlockSpec((1, O), lambda b: (0, 0)),
            pl.BlockSpec((1, O), lambda b: (0, 0)),
        ],
        out_specs=[
            pl.BlockSpec((1, n, O), lambda b: (b, 0, 0)),
            pl.BlockSpec((1, O, n), lambda b: (b, 0, 0)),
        ],
        out_shape=[jax.ShapeDtypeStruct((B, n, O), jnp.float32),
                   jax.ShapeDtypeStruct((B, O, n), jnp.float32)],
    )(mx, mn, sp, qp, gam.reshape(1, O), bet.reshape(1, O))


def _edge_layer(xt, xn, w2T, gam, bet, groups=2, slope=0.2):
    B, n, C = xt.shape
    idx = _knn(xt, xn)
    g = _sc_gather(xt.reshape(B * n, C), idx.reshape(B * n * K))
    g = g.reshape(B, n * K, C)
    mx, mn, sp, qp = _econv(g, xt, w2T)
    return _finalize(mx, mn, sp, qp, gam, bet, groups, slope)


def _gn_rows(t, gam, bet, groups, cnt):
    col_s = jnp.sum(t, axis=0, keepdims=True)
    col_q = jnp.sum(t * t, axis=0, keepdims=True)
    mean_c, var_c = _group_norm_consts(col_s, col_q, cnt, groups)
    return (t - mean_c) / jnp.sqrt(var_c + G_EPS) * gam + bet


def _head_body(x1_ref, x2_ref, x3_ref,
               mlp1_ref, mlp1b_ref, gnm_g_ref, gnm_b_ref,
               c1_ref, c1b_ref, bn1_g_ref, bn1_b_ref,
               c2_ref, c2b_ref, bn2_g_ref, bn2_b_ref,
               s1_ref, s1b_ref, bns_g_ref, bns_b_ref,
               s2_ref, s2b_ref,
               p1_ref, p1b_ref, bnp_g_ref, bnp_b_ref,
               p2_ref, p2b_ref,
               emb_ref, lp_ref):
    n = x1_ref.shape[1]
    xf = jnp.concatenate([x1_ref[0], x2_ref[0], x3_ref[0]], axis=1)
    h = jnp.dot(xf, mlp1_ref[...], preferred_element_type=jnp.float32) \
        + mlp1b_ref[...]
    h = _gn_rows(h, gnm_g_ref[...], gnm_b_ref[...], 8, float(n * 128))
    h = jnp.maximum(h, 0.0)
    x4 = jnp.max(h, axis=0, keepdims=True)
    hh = jnp.concatenate([jnp.broadcast_to(x4, (n, 1024)), xf], axis=1)
    h1 = jnp.dot(hh, c1_ref[...], preferred_element_type=jnp.float32) \
        + c1b_ref[...]
    h1 = _gn_rows(h1, bn1_g_ref[...], bn1_b_ref[...], 8, float(n * 64))
    h1 = jnp.maximum(h1, 0.0)
    xa = jnp.dot(h1, c2_ref[...], preferred_element_type=jnp.float32) \
        + c2b_ref[...]
    xa = _gn_rows(xa, bn2_g_ref[...], bn2_b_ref[...], 4, float(n * 64))
    xa = jnp.maximum(xa, 0.0)
    e = jnp.dot(xa, s1_ref[...], preferred_element_type=jnp.float32) \
        + s1b_ref[...]
    e = _gn_rows(e, bns_g_ref[...], bns_b_ref[...], 4, float(n * 64))
    e = jnp.maximum(e, 0.0)
    emb_ref[0] = jnp.dot(e, s2_ref[...], preferred_element_type=jnp.float32) \
        + s2b_ref[...]
    q = jnp.dot(xa, p1_ref[...], preferred_element_type=jnp.float32) \
        + p1b_ref[...]
    q = _gn_rows(q, bnp_g_ref[...], bnp_b_ref[...], 4, float(n * 64))
    q = jnp.maximum(q, 0.0)
    logits = jnp.dot(q, p2_ref[...], preferred_element_type=jnp.float32) \
        + p2b_ref[...]
    lm = jnp.max(logits, axis=1, keepdims=True)
    shifted = logits - lm
    lp_ref[0] = shifted - jnp.log(jnp.sum(jnp.exp(shifted), axis=1,
                                          keepdims=True))


def _head(x1t, x2t, x3t, p):
    B, n, _ = x1t.shape
    r = lambda v: v.reshape(1, -1)
    const = lambda shp: pl.BlockSpec(shp, lambda b: tuple(0 for _ in shp))
    args = [
        p['mlp1_w'].T, r(p['mlp1_b']), r(p['gnm_g']), r(p['gnm_b']),
        p['c1_w'].T, r(p['c1_b']), r(p['bn1_g']), r(p['bn1_b']),
        p['c2_w'].T, r(p['c2_b']), r(p['bn2_g']), r(p['bn2_b']),
        p['seg1_w'].T, r(p['seg1_b']), r(p['bnseg_g']), r(p['bnseg_b']),
        p['seg2_w'].T, r(p['seg2_b']),
        p['prim1_w'].T, r(p['prim1_b']), r(p['bnprim_g']), r(p['bnprim_b']),
        p['prim2_w'].T, r(p['prim2_b']),
    ]
    in_specs = [
        pl.BlockSpec((1, n, 64), lambda b: (b, 0, 0)),
        pl.BlockSpec((1, n, 64), lambda b: (b, 0, 0)),
        pl.BlockSpec((1, n, 128), lambda b: (b, 0, 0)),
    ] + [const(a.shape) for a in args]
    return pl.pallas_call(
        _head_body,
        grid=(B,),
        in_specs=in_specs,
        out_specs=[
            pl.BlockSpec((1, n, 50), lambda b: (b, 0, 0)),
            pl.BlockSpec((1, n, 8), lambda b: (b, 0, 0)),
        ],
        out_shape=[jax.ShapeDtypeStruct((B, n, 50), jnp.float32),
                   jax.ShapeDtypeStruct((B, n, 8), jnp.float32)],
    )(x1t, x2t, x3t, *args)


def kernel(x, conv1_w, gn1_g, gn1_b, conv2_w, gn2_g, gn2_b, conv3_w, gn3_g,
           gn3_b, mlp1_w, mlp1_b, gnm_g, gnm_b, c1_w, c1_b, bn1_g, bn1_b,
           c2_w, c2_b, bn2_g, bn2_b, seg1_w, seg1_b, bnseg_g, bnseg_b,
           seg2_w, seg2_b, prim1_w, prim1_b, bnprim_g, bnprim_b,
           prim2_w, prim2_b):
    B, C0, n = x.shape
    CP = 16
    xn0 = jnp.concatenate([x, jnp.zeros((B, CP - C0, n), jnp.float32)], axis=1)
    xt0 = jnp.transpose(xn0, (0, 2, 1))
    zpad = jnp.zeros((64, CP - C0), jnp.float32)
    w1 = jnp.concatenate([conv1_w[:, :C0], zpad,
                          conv1_w[:, C0:], zpad], axis=1).T
    x1t, x1n = _edge_layer(xt0, xn0, w1, gn1_g, gn1_b)
    x2t, x2n = _edge_layer(x1t, x1n, conv2_w.T, gn2_g, gn2_b)
    x3t, _ = _edge_layer(x2t, x2n, conv3_w.T, gn3_g, gn3_b)
    p = dict(mlp1_w=mlp1_w, mlp1_b=mlp1_b, gnm_g=gnm_g, gnm_b=gnm_b,
             c1_w=c1_w, c1_b=c1_b, bn1_g=bn1_g, bn1_b=bn1_b,
             c2_w=c2_w, c2_b=c2_b, bn2_g=bn2_g, bn2_b=bn2_b,
             seg1_w=seg1_w, seg1_b=seg1_b, bnseg_g=bnseg_g, bnseg_b=bnseg_b,
             seg2_w=seg2_w, seg2_b=seg2_b,
             prim1_w=prim1_w, prim1_b=prim1_b, bnprim_g=bnprim_g,
             bnprim_b=bnprim_b, prim2_w=prim2_w, prim2_b=prim2_b)
    emb, lp = _head(x1t, x2t, x3t, p)
    return (jnp.transpose(emb, (0, 2, 1)), jnp.transpose(lp, (0, 2, 1)))


# speedup vs baseline: 1.3816x; 1.0488x over previous
import functools
import numpy as np
import jax
import jax.numpy as jnp
from jax import lax
from jax.experimental import pallas as pl
from jax.experimental.pallas import tpu as pltpu
from jax.experimental.pallas import tpu_sc as plsc

K = 80
G_EPS = 1e-5
N = 2048
TN = 2048
TNB = 256
NEG_INF = float('-inf')


def _knn_body(xt_ref, xn_ref, idx_ref, d_ref, iscr_ref):
    a = xt_ref[0]
    xn = xn_ref[0]
    t = pl.program_id(1)
    a_cn = xn_ref[0, :, pl.ds(t * TN, TN)]
    ip = lax.dot_general(a_cn, xn, (((0,), (0,)), ((), ())),
                         preferred_element_type=jnp.float32)
    xx_t = jnp.sum(a * a, axis=1, keepdims=True)
    xx_r = jnp.sum(xn * xn, axis=0, keepdims=True)
    d_ref[...] = 2.0 * ip - xx_t - xx_r

    iota = lax.broadcasted_iota(jnp.int32, (a.shape[0], xn.shape[1]), 1)
    kiota = lax.broadcasted_iota(jnp.int32, (a.shape[0], K), 1)
    iscr_ref[...] = jnp.zeros((a.shape[0], K), jnp.int32)

    def body(k, rm):
        d = d_ref[...]
        am = jnp.min(jnp.where(d >= rm, iota, jnp.int32(N)), axis=1,
                     keepdims=True)
        iscr_ref[...] = iscr_ref[...] + jnp.where(kiota == k, am, 0)
        sel = (iota == am)
        dn = jnp.where(sel, NEG_INF, d)
        d_ref[...] = dn
        return jnp.max(dn, axis=1, keepdims=True)

    rm0 = jnp.max(d_ref[...], axis=1, keepdims=True)
    lax.fori_loop(0, K, body, rm0)
    idx_ref[0] = iscr_ref[...] + pl.program_id(0) * N


def _knn(xt, xn):
    B, n, C = xt.shape
    return pl.pallas_call(
        _knn_body,
        grid=(B, n // TN),
        in_specs=[
            pl.BlockSpec((1, TN, C), lambda b, t: (b, t, 0)),
            pl.BlockSpec((1, C, n), lambda b, t: (b, 0, 0)),
        ],
        out_specs=pl.BlockSpec((1, TN, K), lambda b, t: (b, t, 0)),
        out_shape=jax.ShapeDtypeStruct((B, n, K), jnp.int32),
        scratch_shapes=[pltpu.VMEM((TN, n), jnp.float32),
                        pltpu.VMEM((TN, K), jnp.int32)],
    )(xt, xn)


def _sc_gather(table, idxflat):
    M = idxflat.shape[0]
    C = table.shape[1]
    W = 128
    idx2 = idxflat.reshape(1, M)
    mesh = plsc.VectorSubcoreMesh(core_axis_name="core",
                                  subcore_axis_name="subcore")

    @functools.partial(
        pl.kernel,
        out_type=jax.ShapeDtypeStruct((M, C), jnp.float32),
        mesh=mesh,
        compiler_params=pltpu.CompilerParams(use_tc_tiling_on_sc=False),
    )
    def kern(x_hbm, i_hbm, o_hbm):
        def body(i_vmem, o_vmem):
            pltpu.sync_copy(x_hbm.at[i_vmem.at[0]], o_vmem)

        pltpu.emit_pipeline(
            body,
            grid=(M // W,),
            in_specs=[pl.BlockSpec((1, W), index_map=lambda i: (0, i))],
            out_specs=[pl.BlockSpec((W, C), index_map=lambda i: (i, 0))],
            core_axis_name=("core", "subcore"),
            dimension_semantics=(pltpu.PARALLEL,),
        )(i_hbm, o_hbm)

    return kern(table, idx2)


def _econv_body(g_ref, xt_ref, w_ref, mx_ref, mn_ref, sp_ref, qp_ref):
    g2 = g_ref[0]
    xb = xt_ref[0]
    C = xb.shape[1]
    xbro = jnp.broadcast_to(xb[:, None, :], (TNB, K, C)).reshape(TNB * K, C)
    fcat = jnp.concatenate([g2 - xbro, xbro], axis=1)
    t = jnp.dot(fcat, w_ref[...], preferred_element_type=jnp.float32)
    O = t.shape[1]
    t3 = t.reshape(TNB, K, O)
    mx_ref[0] = jnp.max(t3, axis=1)
    mn_ref[0] = jnp.min(t3, axis=1)
    sp_ref[0, 0] = jnp.sum(t, axis=0, keepdims=True)
    qp_ref[0, 0] = jnp.sum(t * t, axis=0, keepdims=True)


def _econv(g, xt, w2T):
    B, n, C = xt.shape
    O = w2T.shape[1]
    NT = n // TNB
    return pl.pallas_call(
        _econv_body,
        grid=(B, NT),
        in_specs=[
            pl.BlockSpec((1, TNB * K, C), lambda b, t: (b, t, 0)),
            pl.BlockSpec((1, TNB, C), lambda b, t: (b, t, 0)),
            pl.BlockSpec((2 * C, O), lambda b, t: (0, 0)),
        ],
        out_specs=[
            pl.BlockSpec((1, TNB, O), lambda b, t: (b, t, 0)),
            pl.BlockSpec((1, TNB, O), lambda b, t: (b, t, 0)),
            pl.BlockSpec((1, 1, 1, O), lambda b, t: (b, t, 0, 0)),
            pl.BlockSpec((1, 1, 1, O), lambda b, t: (b, t, 0, 0)),
        ],
        out_shape=[jax.ShapeDtypeStruct((B, n, O), jnp.float32),
                   jax.ShapeDtypeStruct((B, n, O), jnp.float32),
                   jax.ShapeDtypeStruct((B, NT, 1, O), jnp.float32),
                   jax.ShapeDtypeStruct((B, NT, 1, O), jnp.float32)],
    )(g, xt, w2T)


def _group_norm_consts(s, q, cnt, G):
    O = s.shape[1]
    W = O // G
    mc, vc = [], []
    for g in range(G):
        sg = jnp.sum(s[:, g * W:(g + 1) * W], axis=1, keepdims=True) / cnt
        qg = jnp.sum(q[:, g * W:(g + 1) * W], axis=1, keepdims=True) / cnt
        mc.append(jnp.broadcast_to(sg, (1, W)))
        vc.append(jnp.broadcast_to(qg - sg * sg, (1, W)))
    return jnp.concatenate(mc, axis=1), jnp.concatenate(vc, axis=1)


def _fin_body(mx_ref, mn_ref, sp_ref, qp_ref, g_ref, b_ref,
              xt_ref, xn_ref, *, cnt, slope, groups):
    s = jnp.sum(sp_ref[0, :, 0, :], axis=0, keepdims=True)
    q = jnp.sum(qp_ref[0, :, 0, :], axis=0, keepdims=True)
    mean_c, var_c = _group_norm_consts(s, q, cnt, groups)
    den = jnp.sqrt(var_c + G_EPS)
    gam = g_ref[...]
    bet = b_ref[...]
    sel_v = jnp.where(gam >= 0, mx_ref[0], mn_ref[0])
    xh = (sel_v - mean_c) / den * gam + bet
    xo = jnp.where(xh >= 0, xh, slope * xh)
    xt_ref[0] = xo
    xn_ref[0] = xo.T


def _finalize(mx, mn, sp, qp, gam, bet, groups, slope):
    B, n, O = mx.shape
    cnt = float((O // groups) * n * K)
    body = functools.partial(_fin_body, cnt=cnt, slope=slope, groups=groups)
    NT = sp.shape[1]
    assert sp.shape == (B, NT, 1, O)
    return pl.pallas_call(
        body,
        grid=(B,),
        in_specs=[
            pl.BlockSpec((1, n, O), lambda b: (b, 0, 0)),
            pl.BlockSpec((1, n, O), lambda b: (b, 0, 0)),
            pl.BlockSpec((1, NT, 1, O), lambda b: (b, 0, 0, 0)),
            pl.BlockSpec((1, NT, 1, O), lambda b: (b, 0, 0, 0)),
            pl.BlockSpec((1, O), lambda b: (0, 0)),
            pl.BlockSpec((1, O), lambda b: (0, 0)),
        ],
        out_specs=[
            pl.BlockSpec((1, n, O), lambda b: (b, 0, 0)),
            pl.BlockSpec((1, O, n), lambda b: (b, 0, 0)),
        ],
        out_shape=[jax.ShapeDtypeStruct((B, n, O), jnp.float32),
                   jax.ShapeDtypeStruct((B, O, n), jnp.float32)],
    )(mx, mn, sp, qp, gam.reshape(1, O), bet.reshape(1, O))


def _edge_layer(xt, xn, w2T, gam, bet, groups=2, slope=0.2):
    B, n, C = xt.shape
    idx = _knn(xt, xn)
    g = _sc_gather(xt.reshape(B * n, C), idx.reshape(B * n * K))
    g = g.reshape(B, n * K, C)
    mx, mn, sp, qp = _econv(g, xt, w2T)
    return _finalize(mx, mn, sp, qp, gam, bet, groups, slope)


def _gn_rows(t, gam, bet, groups, cnt):
    col_s = jnp.sum(t, axis=0, keepdims=True)
    col_q = jnp.sum(t * t, axis=0, keepdims=True)
    mean_c, var_c = _group_norm_consts(col_s, col_q, cnt, groups)
    return (t - mean_c) / jnp.sqrt(var_c + G_EPS) * gam + bet


def _head_body(x1_ref, x2_ref, x3_ref,
               mlp1_ref, mlp1b_ref, gnm_g_ref, gnm_b_ref,
               c1_ref, c1b_ref, bn1_g_ref, bn1_b_ref,
               c2_ref, c2b_ref, bn2_g_ref, bn2_b_ref,
               s1_ref, s1b_ref, bns_g_ref, bns_b_ref,
               s2_ref, s2b_ref,
               p1_ref, p1b_ref, bnp_g_ref, bnp_b_ref,
               p2_ref, p2b_ref,
               emb_ref, lp_ref):
    n = x1_ref.shape[1]
    xf = jnp.concatenate([x1_ref[0], x2_ref[0], x3_ref[0]], axis=1)
    h = jnp.dot(xf, mlp1_ref[...], preferred_element_type=jnp.float32) \
        + mlp1b_ref[...]
    h = _gn_rows(h, gnm_g_ref[...], gnm_b_ref[...], 8, float(n * 128))
    h = jnp.maximum(h, 0.0)
    x4 = jnp.max(h, axis=0, keepdims=True)
    hh = jnp.concatenate([jnp.broadcast_to(x4, (n, 1024)), xf], axis=1)
    h1 = jnp.dot(hh, c1_ref[...], preferred_element_type=jnp.float32) \
        + c1b_ref[...]
    h1 = _gn_rows(h1, bn1_g_ref[...], bn1_b_ref[...], 8, float(n * 64))
    h1 = jnp.maximum(h1, 0.0)
    xa = jnp.dot(h1, c2_ref[...], preferred_element_type=jnp.float32) \
        + c2b_ref[...]
    xa = _gn_rows(xa, bn2_g_ref[...], bn2_b_ref[...], 4, float(n * 64))
    xa = jnp.maximum(xa, 0.0)
    e = jnp.dot(xa, s1_ref[...], preferred_element_type=jnp.float32) \
        + s1b_ref[...]
    e = _gn_rows(e, bns_g_ref[...], bns_b_ref[...], 4, float(n * 64))
    e = jnp.maximum(e, 0.0)
    emb_ref[0] = jnp.dot(e, s2_ref[...], preferred_element_type=jnp.float32) \
        + s2b_ref[...]
    q = jnp.dot(xa, p1_ref[...], preferred_element_type=jnp.float32) \
        + p1b_ref[...]
    q = _gn_rows(q, bnp_g_ref[...], bnp_b_ref[...], 4, float(n * 64))
    q = jnp.maximum(q, 0.0)
    logits = jnp.dot(q, p2_ref[...], preferred_element_type=jnp.float32) \
        + p2b_ref[...]
    lm = jnp.max(logits, axis=1, keepdims=True)
    shifted = logits - lm
    lp_ref[0] = shifted - jnp.log(jnp.sum(jnp.exp(shifted), axis=1,
                                          keepdims=True))


def _head(x1t, x2t, x3t, p):
    B, n, _ = x1t.shape
    r = lambda v: v.reshape(1, -1)
    const = lambda shp: pl.BlockSpec(shp, lambda b: tuple(0 for _ in shp))
    args = [
        p['mlp1_w'].T, r(p['mlp1_b']), r(p['gnm_g']), r(p['gnm_b']),
        p['c1_w'].T, r(p['c1_b']), r(p['bn1_g']), r(p['bn1_b']),
        p['c2_w'].T, r(p['c2_b']), r(p['bn2_g']), r(p['bn2_b']),
        p['seg1_w'].T, r(p['seg1_b']), r(p['bnseg_g']), r(p['bnseg_b']),
        p['seg2_w'].T, r(p['seg2_b']),
        p['prim1_w'].T, r(p['prim1_b']), r(p['bnprim_g']), r(p['bnprim_b']),
        p['prim2_w'].T, r(p['prim2_b']),
    ]
    in_specs = [
        pl.BlockSpec((1, n, 64), lambda b: (b, 0, 0)),
        pl.BlockSpec((1, n, 64), lambda b: (b, 0, 0)),
        pl.BlockSpec((1, n, 128), lambda b: (b, 0, 0)),
    ] + [const(a.shape) for a in args]
    return pl.pallas_call(
        _head_body,
        grid=(B,),
        in_specs=in_specs,
        out_specs=[
            pl.BlockSpec((1, n, 50), lambda b: (b, 0, 0)),
            pl.BlockSpec((1, n, 8), lambda b: (b, 0, 0)),
        ],
        out_shape=[jax.ShapeDtypeStruct((B, n, 50), jnp.float32),
                   jax.ShapeDtypeStruct((B, n, 8), jnp.float32)],
    )(x1t, x2t, x3t, *args)


def kernel(x, conv1_w, gn1_g, gn1_b, conv2_w, gn2_g, gn2_b, conv3_w, gn3_g,
           gn3_b, mlp1_w, mlp1_b, gnm_g, gnm_b, c1_w, c1_b, bn1_g, bn1_b,
           c2_w, c2_b, bn2_g, bn2_b, seg1_w, seg1_b, bnseg_g, bnseg_b,
           seg2_w, seg2_b, prim1_w, prim1_b, bnprim_g, bnprim_b,
           prim2_w, prim2_b):
    B, C0, n = x.shape
    CP = 16
    xn0 = jnp.concatenate([x, jnp.zeros((B, CP - C0, n), jnp.float32)], axis=1)
    xt0 = jnp.transpose(xn0, (0, 2, 1))
    zpad = jnp.zeros((64, CP - C0), jnp.float32)
    w1 = jnp.concatenate([conv1_w[:, :C0], zpad,
                          conv1_w[:, C0:], zpad], axis=1).T
    x1ts, x2ts, x3ts = [], [], []
    for b in range(B):
        x1t_b, x1n_b = _edge_layer(xt0[b:b + 1], xn0[b:b + 1], w1,
                                   gn1_g, gn1_b)
        x2t_b, x2n_b = _edge_layer(x1t_b, x1n_b, conv2_w.T, gn2_g, gn2_b)
        x3t_b, _ = _edge_layer(x2t_b, x2n_b, conv3_w.T, gn3_g, gn3_b)
        x1ts.append(x1t_b)
        x2ts.append(x2t_b)
        x3ts.append(x3t_b)
    x1t = jnp.concatenate(x1ts, axis=0)
    x2t = jnp.concatenate(x2ts, axis=0)
    x3t = jnp.concatenate(x3ts, axis=0)
    p = dict(mlp1_w=mlp1_w, mlp1_b=mlp1_b, gnm_g=gnm_g, gnm_b=gnm_b,
             c1_w=c1_w, c1_b=c1_b, bn1_g=bn1_g, bn1_b=bn1_b,
             c2_w=c2_w, c2_b=c2_b, bn2_g=bn2_g, bn2_b=bn2_b,
             seg1_w=seg1_w, seg1_b=seg1_b, bnseg_g=bnseg_g, bnseg_b=bnseg_b,
             seg2_w=seg2_w, seg2_b=seg2_b,
             prim1_w=prim1_w, prim1_b=prim1_b, bnprim_g=bnprim_g,
             bnprim_b=bnprim_b, prim2_w=prim2_w, prim2_b=prim2_b)
    emb, lp = _head(x1t, x2t, x3t, p)
    return (jnp.transpose(emb, (0, 2, 1)), jnp.transpose(lp, (0, 2, 1)))
